# baseline (speedup 1.0000x reference)
import jax
import jax.numpy as jnp
from jax import lax
from jax.experimental import pallas as pl
from jax.experimental.pallas import tpu as pltpu
from jax.experimental.pallas import tpu_sc as plsc

_B = 16384
_E = 64
_NC = 2
_NS = 16
_NW = _NC * _NS
_BPW = _B // _NW
_CH = 128
_NCH = _BPW // _CH
_NAL = 4


def _body(user_h, course_h, uta, utb, utc, utd, ctp_h, wb_h, out_h,
          uidx, cidx, cpidx, u0, u1, cb0, cb1, cb2, cb3, wbv, outv, *sems):
    cid = lax.axis_index("c")
    sid = lax.axis_index("s")
    wid = sid * _NC + cid
    base = wid * _BPW

    uts = (uta, utb, utc, utd)
    ubufs = (u0, u1)
    cbufs = (cb0, cb1, cb2, cb3)
    csems = sems[:_NCH]
    usems = sems[_NCH:]

    pltpu.sync_copy(wb_h, wbv)
    pltpu.sync_copy(user_h.at[pl.ds(base, _BPW)], uidx)
    pltpu.sync_copy(course_h.at[pl.ds(base, _BPW)], cidx)

    def p_body(g, carry):
        off = pl.multiple_of(g * 16, 16)
        cpidx[pl.ds(off, 16)] = lax.shift_right_logical(
            cidx[pl.ds(off, 16)], 1)
        return carry
    lax.fori_loop(0, _BPW // 16, p_body, 0)

    hc = {}
    for ch in range(_NCH):
        hc[ch] = pltpu.async_copy(
            ctp_h.at[cpidx.at[pl.ds(ch * _CH, _CH)]], cbufs[ch], csems[ch])

    def u_src(lane, r):
        return uts[lane % _NAL].at[r, :]

    def issue_chunk(ch):
        ub = ubufs[ch % 2]

        def g_body(g, carry):
            off = ch * _CH + g * 16
            iu = uidx[pl.ds(off, 16)]
            dst = g * 16
            for lane in range(1):
                pltpu.async_copy(
                    u_src(lane, jnp.int32(7)),
                    ub.at[dst + lane, :],
                    usems[(ch % 2) * _NAL + lane % _NAL])
            return carry
        lax.fori_loop(0, _CH // 16, g_body, 0)

    def drain_chunk(ch):
        ub = ubufs[ch % 2]

        def d_body(g, carry):
            dst = g * 16
            for lane in range(1):
                pltpu.make_async_copy(
                    u_src(lane, jnp.int32(0)),
                    ub.at[dst + lane, :],
                    usems[(ch % 2) * _NAL + lane % _NAL]).wait()
            return carry
        lax.fori_loop(0, _CH // 16, d_body, 0)

    wvecs = [wbv[pl.ds(k * 16, 16)] for k in range(5)]
    bias = wvecs[4][0]
    lane_iota = lax.iota(jnp.int32, 16)
    one = jnp.full((16,), 1, jnp.int32)

    def compute_chunk(ch):
        ub = ubufs[ch % 2]
        cb = cbufs[ch]

        def g_body(g, carry):
            r0 = ch * _CH + g * 16
            rows = g * 16 + lane_iota
            vc = cidx[pl.ds(pl.multiple_of(r0, 16), 16)]
            ccolb = lax.shift_left(lax.bitwise_and(vc, one), 6)
            accs = [jnp.zeros((16,), jnp.float32) + bias,
                    jnp.zeros((16,), jnp.float32),
                    jnp.zeros((16,), jnp.float32),
                    jnp.zeros((16,), jnp.float32)]
            for j in range(_E):
                jv = jnp.full((16,), j, jnp.int32)
                uu = plsc.load_gather(ub, [rows, jv])
                cc = plsc.load_gather(cb, [rows, ccolb + j])
                accs[j % 4] = accs[j % 4] + uu * cc * wvecs[j // 16][j % 16]
            acc = (accs[0] + accs[1]) + (accs[2] + accs[3])
            outv[pl.ds(pl.multiple_of(r0, 16), 16)] = acc
            return carry
        lax.fori_loop(0, _CH // 16, g_body, 0)

    issue_chunk(0)
    issue_chunk(1)
    for ch in range(_NCH):
        drain_chunk(ch)
        hc[ch].wait()
        compute_chunk(ch)
        if ch + 2 < _NCH:
            issue_chunk(ch + 2)

    pltpu.sync_copy(outv, out_h.at[pl.ds(base, _BPW)])


def _run(user, course, user_table, ct_pair, wb):
    mesh = plsc.VectorSubcoreMesh(core_axis_name="c", subcore_axis_name="s")
    f = pl.kernel(
        _body,
        mesh=mesh,
        compiler_params=pltpu.CompilerParams(needs_layout_passes=False),
        out_type=jax.ShapeDtypeStruct((_B,), jnp.float32),
        scratch_types=[
            pltpu.VMEM((_BPW,), jnp.int32),
            pltpu.VMEM((_BPW,), jnp.int32),
            pltpu.VMEM((_BPW,), jnp.int32),
            pltpu.VMEM((_CH, _E), jnp.float32),
            pltpu.VMEM((_CH, _E), jnp.float32),
            pltpu.VMEM((_CH, 2 * _E), jnp.float32),
            pltpu.VMEM((_CH, 2 * _E), jnp.float32),
            pltpu.VMEM((_CH, 2 * _E), jnp.float32),
            pltpu.VMEM((_CH, 2 * _E), jnp.float32),
            pltpu.VMEM((80,), jnp.float32),
            pltpu.VMEM((_BPW,), jnp.float32),
        ] + [pltpu.SemaphoreType.DMA] * (_NCH + 2 * _NAL),
    )
    return f(user, course, user_table, user_table, user_table, user_table,
             ct_pair, wb)


def kernel(user, course, user_table, course_table, fc_w, fc_b):
    wb = jnp.zeros((80,), jnp.float32)
    wb = wb.at[:_E].set(fc_w.reshape(-1)).at[_E].set(fc_b[0])
    ct_pair = course_table.reshape(course_table.shape[0] // 2, 2 * _E)
    out = _run(user, course, user_table, ct_pair, wb)
    return out.reshape(_B, 1)


# speedup vs baseline: 1.0581x; 1.0581x over previous
import jax
import jax.numpy as jnp
from jax import lax
from jax.experimental import pallas as pl
from jax.experimental.pallas import tpu as pltpu
from jax.experimental.pallas import tpu_sc as plsc

_B = 16384
_E = 64
_NC = 2
_NS = 16
_NW = _NC * _NS
_BPW = _B // _NW
_CH = 128
_NCH = _BPW // _CH
_NAL = 4


def _body(user_h, course_h, uta, utb, utc, utd, ctp_h, wb_h, out_h,
          uidx, cidx, cpidx, u0, u1, cb0, cb1, cb2, cb3, wbv, outv, *sems):
    cid = lax.axis_index("c")
    sid = lax.axis_index("s")
    wid = sid * _NC + cid
    base = wid * _BPW

    uts = (uta, utb, utc, utd)
    ubufs = (u0, u1)
    cbufs = (cb0, cb1, cb2, cb3)
    csems = sems[:_NCH]
    usems = sems[_NCH:]

    pltpu.sync_copy(wb_h, wbv)
    pltpu.sync_copy(user_h.at[pl.ds(base, _BPW)], uidx)
    pltpu.sync_copy(course_h.at[pl.ds(base, _BPW)], cidx)

    def p_body(g, carry):
        off = pl.multiple_of(g * 16, 16)
        cpidx[pl.ds(off, 16)] = lax.shift_right_logical(
            cidx[pl.ds(off, 16)], 1)
        return carry
    lax.fori_loop(0, _BPW // 16, p_body, 0)

    hc = {}
    for ch in range(_NCH):
        hc[ch] = pltpu.async_copy(
            ctp_h.at[cpidx.at[pl.ds(ch * _CH, _CH)]], cbufs[ch], csems[ch])

    def u_src(lane, r):
        return uts[lane % _NAL].at[r, :]

    def issue_chunk(ch):
        ub = ubufs[ch % 2]

        def g_body(g, carry):
            off = ch * _CH + g * 16
            iu = uidx[pl.ds(off, 16)]
            dst = g * 16
            return carry
        lax.fori_loop(0, _CH // 16, g_body, 0)

    def drain_chunk(ch):
        ub = ubufs[ch % 2]

        def d_body(g, carry):
            dst = g * 16
            return carry
        lax.fori_loop(0, _CH // 16, d_body, 0)

    wvecs = [wbv[pl.ds(k * 16, 16)] for k in range(5)]
    bias = wvecs[4][0]
    lane_iota = lax.iota(jnp.int32, 16)
    one = jnp.full((16,), 1, jnp.int32)

    def compute_chunk(ch):
        ub = ubufs[ch % 2]
        cb = cbufs[ch]

        def g_body(g, carry):
            r0 = ch * _CH + g * 16
            rows = g * 16 + lane_iota
            vc = cidx[pl.ds(pl.multiple_of(r0, 16), 16)]
            ccolb = lax.shift_left(lax.bitwise_and(vc, one), 6)
            accs = [jnp.zeros((16,), jnp.float32) + bias,
                    jnp.zeros((16,), jnp.float32),
                    jnp.zeros((16,), jnp.float32),
                    jnp.zeros((16,), jnp.float32)]
            for j in range(_E):
                jv = jnp.full((16,), j, jnp.int32)
                uu = plsc.load_gather(ub, [rows, jv])
                cc = plsc.load_gather(cb, [rows, ccolb + j])
                accs[j % 4] = accs[j % 4] + uu * cc * wvecs[j // 16][j % 16]
            acc = (accs[0] + accs[1]) + (accs[2] + accs[3])
            outv[pl.ds(pl.multiple_of(r0, 16), 16)] = acc
            return carry
        lax.fori_loop(0, _CH // 16, g_body, 0)

    issue_chunk(0)
    issue_chunk(1)
    for ch in range(_NCH):
        drain_chunk(ch)
        hc[ch].wait()
        compute_chunk(ch)
        if ch + 2 < _NCH:
            issue_chunk(ch + 2)

    pltpu.sync_copy(outv, out_h.at[pl.ds(base, _BPW)])


def _run(user, course, user_table, ct_pair, wb):
    mesh = plsc.VectorSubcoreMesh(core_axis_name="c", subcore_axis_name="s")
    f = pl.kernel(
        _body,
        mesh=mesh,
        compiler_params=pltpu.CompilerParams(needs_layout_passes=False),
        out_type=jax.ShapeDtypeStruct((_B,), jnp.float32),
        scratch_types=[
            pltpu.VMEM((_BPW,), jnp.int32),
            pltpu.VMEM((_BPW,), jnp.int32),
            pltpu.VMEM((_BPW,), jnp.int32),
            pltpu.VMEM((_CH, _E), jnp.float32),
            pltpu.VMEM((_CH, _E), jnp.float32),
            pltpu.VMEM((_CH, 2 * _E), jnp.float32),
            pltpu.VMEM((_CH, 2 * _E), jnp.float32),
            pltpu.VMEM((_CH, 2 * _E), jnp.float32),
            pltpu.VMEM((_CH, 2 * _E), jnp.float32),
            pltpu.VMEM((80,), jnp.float32),
            pltpu.VMEM((_BPW,), jnp.float32),
        ] + [pltpu.SemaphoreType.DMA] * (_NCH + 2 * _NAL),
    )
    return f(user, course, user_table, user_table, user_table, user_table,
             ct_pair, wb)


def kernel(user, course, user_table, course_table, fc_w, fc_b):
    wb = jnp.zeros((80,), jnp.float32)
    wb = wb.at[:_E].set(fc_w.reshape(-1)).at[_E].set(fc_b[0])
    ct_pair = course_table.reshape(course_table.shape[0] // 2, 2 * _E)
    out = _run(user, course, user_table, ct_pair, wb)
    return out.reshape(_B, 1)


# speedup vs baseline: 1.0791x; 1.0199x over previous
import jax
import jax.numpy as jnp
from jax import lax
from jax.experimental import pallas as pl
from jax.experimental.pallas import tpu as pltpu
from jax.experimental.pallas import tpu_sc as plsc

_B = 16384
_E = 64
_NC = 2
_NS = 16
_NW = _NC * _NS
_BPW = _B // _NW
_CH = 128
_NCH = _BPW // _CH


def _body(user_h, course_h, ut_h, ct_h, wb_h, out_h,
          uidx, cidx, u0, u1, c0, c1, wbv, outv, *sems):
    cid = lax.axis_index("c")
    sid = lax.axis_index("s")
    wid = sid * _NC + cid
    base = wid * _BPW

    ubufs = (u0, u1)
    cbufs = (c0, c1)

    pltpu.sync_copy(wb_h, wbv)
    pltpu.sync_copy(user_h.at[pl.ds(base, _BPW)], uidx)
    pltpu.sync_copy(course_h.at[pl.ds(base, _BPW)], cidx)

    def issue_chunk(ch):
        ub = ubufs[ch % 2]
        cb = cbufs[ch % 2]
        us = sems[ch % 2]
        cs = sems[2 + ch % 2]

        def r_body(r, carry):
            rv = r + ch * _CH
            ridx = jnp.zeros((16,), jnp.int32) + rv
            ru = plsc.load_gather(uidx, [ridx])[0]
            rc = plsc.load_gather(cidx, [ridx])[0]
            pltpu.async_copy(ut_h.at[ru, :], ub.at[r, :], us)
            pltpu.async_copy(ct_h.at[rc, :], cb.at[r, :], cs)
            return carry
        lax.fori_loop(0, _CH, r_body, 0)

    def drain_chunk(ch):
        ub = ubufs[ch % 2]
        cb = cbufs[ch % 2]
        us = sems[ch % 2]
        cs = sems[2 + ch % 2]

        def r_body(r, carry):
            pltpu.make_async_copy(ut_h.at[0, :], ub.at[0, :], us).wait()
            pltpu.make_async_copy(ct_h.at[0, :], cb.at[0, :], cs).wait()
            return carry
        lax.fori_loop(0, _CH, r_body, 0)

    wvecs = [wbv[pl.ds(k * 16, 16)] for k in range(5)]
    bias = wvecs[4][0]
    lane_iota = lax.iota(jnp.int32, 16)
    zero16 = jnp.zeros((16,), jnp.int32)

    def compute_chunk(ch):
        ub = ubufs[ch % 2]
        cb = cbufs[ch % 2]

        def g_body(g, carry):
            rows = g * 16 + lane_iota

            def jj_body(jj, accs):
                jb = jj * 16
                wv = wbv[pl.ds(pl.multiple_of(jb, 16), 16)]
                ibase = zero16 + jb
                for t in range(16):
                    col = ibase + t
                    uu = plsc.load_gather(ub, [rows, col])
                    cc = plsc.load_gather(cb, [rows, col])
                    accs[t % 4] = accs[t % 4] + uu * cc * wv[t]
                return accs

            accs = lax.fori_loop(0, _E // 16, jj_body,
                                 [jnp.zeros((16,), jnp.float32) + bias,
                                  jnp.zeros((16,), jnp.float32),
                                  jnp.zeros((16,), jnp.float32),
                                  jnp.zeros((16,), jnp.float32)])
            acc = (accs[0] + accs[1]) + (accs[2] + accs[3])
            off = pl.multiple_of(ch * _CH + g * 16, 16)
            outv[pl.ds(off, 16)] = acc
            return carry
        lax.fori_loop(0, _CH // 16, g_body, 0)

    issue_chunk(0)
    issue_chunk(1)
    for ch in range(_NCH):
        drain_chunk(ch)
        compute_chunk(ch)
        if ch + 2 < _NCH:
            issue_chunk(ch + 2)

    pltpu.sync_copy(outv, out_h.at[pl.ds(base, _BPW)])


def _run(user, course, user_table, course_table, wb):
    mesh = plsc.VectorSubcoreMesh(core_axis_name="c", subcore_axis_name="s")
    f = pl.kernel(
        _body,
        mesh=mesh,
        compiler_params=pltpu.CompilerParams(needs_layout_passes=False),
        out_type=jax.ShapeDtypeStruct((_B,), jnp.float32),
        scratch_types=[
            pltpu.VMEM((_BPW,), jnp.int32),
            pltpu.VMEM((_BPW,), jnp.int32),
            pltpu.VMEM((_CH, _E), jnp.float32),
            pltpu.VMEM((_CH, _E), jnp.float32),
            pltpu.VMEM((_CH, _E), jnp.float32),
            pltpu.VMEM((_CH, _E), jnp.float32),
            pltpu.VMEM((80,), jnp.float32),
            pltpu.VMEM((_BPW,), jnp.float32),
        ] + [pltpu.SemaphoreType.DMA] * 4,
    )
    return f(user, course, user_table, course_table, wb)


def kernel(user, course, user_table, course_table, fc_w, fc_b):
    wb = jnp.zeros((80,), jnp.float32)
    wb = wb.at[:_E].set(fc_w.reshape(-1)).at[_E].set(fc_b[0])
    out = _run(user, course, user_table, course_table, wb)
    return out.reshape(_B, 1)
